# baseline (speedup 1.0000x reference)
import functools

import jax
import jax.numpy as jnp
from jax import lax
from jax.experimental import pallas as pl
from jax.experimental.pallas import tpu as pltpu
from jax.experimental.pallas import tpu_sc as plsc

T = 2048
H = 1024
F = 2048
E = 8
CAP = 384
S = E * CAP
TEMP = 0.8
FB = 1024
NFB = F // FB

NC = 2
NS = 16
NW = NC * NS
LANES = 16

ROWS_C = T // NW


def _router_body(wg_ref, x_ref, bg_ref, slot_ref, gs_ref):
    z = lax.dot_general(
        wg_ref[...], x_ref[...],
        dimension_numbers=(((1,), (1,)), ((), ())),
        preferred_element_type=jnp.float32,
    )
    z = (z + bg_ref[...]) / TEMP
    zmax = jnp.max(z, axis=0, keepdims=True)
    p = jnp.exp(z - zmax)
    ssum = jnp.sum(p, axis=0, keepdims=True)
    probs = p / ssum
    gate = jnp.max(probs, axis=0, keepdims=True)
    iota_e = lax.broadcasted_iota(jnp.int32, (E, T), 0)
    idx = jnp.min(jnp.where(probs == gate, iota_e, E), axis=0, keepdims=True)
    onehot = (iota_e == idx).astype(jnp.float32)
    c = onehot
    k = 1
    while k < T:
        c = c + jnp.concatenate(
            [jnp.zeros((E, k), jnp.float32), c[:, :-k]], axis=1)
        k *= 2
    pos = jnp.sum(onehot * c, axis=0, keepdims=True) - 1.0
    kept = pos < CAP
    counts = jnp.sum(onehot, axis=1, keepdims=True)
    cmin = jnp.min(counts, axis=0, keepdims=True)
    iota_ec = lax.broadcasted_iota(jnp.int32, (E, 1), 0)
    emin = jnp.min(jnp.where(counts == cmin, iota_ec, E), axis=0,
                   keepdims=True)
    trash = emin * CAP + (CAP - 1)
    posi = pos.astype(jnp.int32)
    slot_ref[...] = jnp.where(kept, idx * CAP + posi, trash)
    iota_c = lax.broadcasted_iota(jnp.int32, (CAP, T), 0)
    p_t = (iota_c == posi).astype(jnp.float32)
    u = onehot * gate
    gs_ref[...] = lax.dot_general(
        u, p_t, (((1,), (1,)), ((), ())),
        precision=lax.Precision.HIGHEST,
        preferred_element_type=jnp.float32)


def _ffn_body(slot_ref, x_ref, w1_ref, w2_ref, w3_ref, g_ref, y_ref, xb_ref):
    e = pl.program_id(0)
    fb = pl.program_id(1)

    @pl.when(fb == 0)
    def _():
        iota_c = lax.broadcasted_iota(jnp.int32, (CAP, T), 0)
        sl = jnp.broadcast_to(slot_ref[...], (CAP, T))
        onehot = (sl == iota_c + e * CAP).astype(jnp.float32)
        xb_ref[...] = lax.dot_general(
            onehot, x_ref[...], (((1,), (0,)), ((), ())),
            preferred_element_type=jnp.float32)

    xb = xb_ref[...]
    h1 = lax.dot_general(xb, w1_ref[0], (((1,), (1,)), ((), ())),
                         preferred_element_type=jnp.float32)
    h2 = lax.dot_general(xb, w2_ref[0], (((1,), (1,)), ((), ())),
                         preferred_element_type=jnp.float32)
    act = h1 * lax.logistic(h1) * h2
    p = lax.dot_general(act, w3_ref[0], (((1,), (1,)), ((), ())),
                        preferred_element_type=jnp.float32)
    p = p * g_ref[0]

    @pl.when(fb == 0)
    def _():
        y_ref[0] = p

    @pl.when(fb > 0)
    def _():
        y_ref[0] = y_ref[0] + p


def _combine_body(y_hbm, slot_hbm, out_hbm, idx_v, rows_v, sem):
    wid = lax.axis_index("s") * NC + lax.axis_index("c")
    base = wid * ROWS_C
    pltpu.sync_copy(slot_hbm.at[pl.ds(base, ROWS_C)], idx_v)
    pltpu.async_copy(y_hbm.at[idx_v], rows_v, sem).wait()
    pltpu.sync_copy(rows_v, out_hbm.at[pl.ds(base, ROWS_C)])


@functools.cache
def _sc_calls():
    mesh = plsc.VectorSubcoreMesh(
        core_axis_name="c", subcore_axis_name="s",
        num_cores=NC, num_subcores=NS)
    combine = functools.partial(
        pl.kernel,
        out_type=[jax.ShapeDtypeStruct((T, H), jnp.float32)],
        mesh=mesh,
        compiler_params=pltpu.CompilerParams(needs_layout_passes=False),
        scratch_types=[
            pltpu.VMEM((ROWS_C,), jnp.int32),
            pltpu.VMEM((ROWS_C, H), jnp.float32),
            pltpu.SemaphoreType.DMA,
        ],
    )(_combine_body)
    return combine


def kernel(x, W1, W2, W3, Wg, bg):
    slot2, gs = pl.pallas_call(
        _router_body,
        out_shape=[
            jax.ShapeDtypeStruct((1, T), jnp.int32),
            jax.ShapeDtypeStruct((E, CAP), jnp.float32),
        ],
    )(Wg, x, bg.reshape(E, 1))
    slot = slot2.reshape(T)

    y = pl.pallas_call(
        _ffn_body,
        grid=(E, NFB),
        in_specs=[
            pl.BlockSpec((1, T), lambda e, f: (0, 0)),
            pl.BlockSpec((T, H), lambda e, f: (0, 0)),
            pl.BlockSpec((1, FB, H), lambda e, f: (e, f, 0)),
            pl.BlockSpec((1, FB, H), lambda e, f: (e, f, 0)),
            pl.BlockSpec((1, H, FB), lambda e, f: (e, 0, f)),
            pl.BlockSpec((1, CAP, 1), lambda e, f: (e, 0, 0)),
        ],
        out_specs=pl.BlockSpec((1, CAP, H), lambda e, f: (e, 0, 0)),
        out_shape=jax.ShapeDtypeStruct((E, CAP, H), jnp.float32),
        scratch_shapes=[pltpu.VMEM((CAP, H), jnp.float32)],
    )(
        slot2, x, W1, W2, W3, gs.reshape(E, CAP, 1)
    )

    _combine_call = _sc_calls()
    (out,) = _combine_call(y.reshape(S, H), slot)
    return out


# speedup vs baseline: 1.0345x; 1.0345x over previous
import functools

import jax
import jax.numpy as jnp
from jax import lax
from jax.experimental import pallas as pl
from jax.experimental.pallas import tpu as pltpu
from jax.experimental.pallas import tpu_sc as plsc

T = 2048
H = 1024
F = 2048
E = 8
CAP = 384
S = E * CAP
TEMP = 0.8
FB = 2048
NFB = F // FB

NC = 2
NS = 16
NW = NC * NS
LANES = 16

ROWS_B = S // NW
ROWS_C = T // NW


def _router_body(wg_ref, x_ref, bg_ref, slot_ref, gs_ref, tok_ref):
    z = lax.dot_general(
        wg_ref[...], x_ref[...],
        dimension_numbers=(((1,), (1,)), ((), ())),
        preferred_element_type=jnp.float32,
    )
    z = (z + bg_ref[...]) / TEMP
    zmax = jnp.max(z, axis=0, keepdims=True)
    p = jnp.exp(z - zmax)
    ssum = jnp.sum(p, axis=0, keepdims=True)
    probs = p / ssum
    gate = jnp.max(probs, axis=0, keepdims=True)
    iota_e = lax.broadcasted_iota(jnp.int32, (E, T), 0)
    idx = jnp.min(jnp.where(probs == gate, iota_e, E), axis=0, keepdims=True)
    onehot = (iota_e == idx).astype(jnp.float32)
    c = onehot
    k = 1
    while k < T:
        c = c + jnp.concatenate(
            [jnp.zeros((E, k), jnp.float32), c[:, :-k]], axis=1)
        k *= 2
    pos = jnp.sum(onehot * c, axis=0, keepdims=True) - 1.0
    kept = pos < CAP
    counts = jnp.sum(onehot, axis=1, keepdims=True)
    cmin = jnp.min(counts, axis=0, keepdims=True)
    iota_ec = lax.broadcasted_iota(jnp.int32, (E, 1), 0)
    emin = jnp.min(jnp.where(counts == cmin, iota_ec, E), axis=0,
                   keepdims=True)
    trash = emin * CAP + (CAP - 1)
    posi = pos.astype(jnp.int32)
    slot_ref[...] = jnp.where(kept, idx * CAP + posi, trash)
    iota_c = lax.broadcasted_iota(jnp.int32, (CAP, T), 0)
    p_t = (iota_c == posi).astype(jnp.float32)
    u = onehot * gate
    gs_ref[...] = lax.dot_general(
        u, p_t, (((1,), (1,)), ((), ())),
        precision=lax.Precision.HIGHEST,
        preferred_element_type=jnp.float32)
    iota_t = lax.broadcasted_iota(jnp.int32, (E, T), 1).astype(jnp.float32)
    v = onehot * iota_t
    tokf = lax.dot_general(
        v, p_t, (((1,), (1,)), ((), ())),
        precision=lax.Precision.HIGHEST,
        preferred_element_type=jnp.float32)
    slot_id = lax.broadcasted_iota(jnp.int32, (E, CAP), 0) * CAP + \
        lax.broadcasted_iota(jnp.int32, (E, CAP), 1)
    filler = lax.rem(slot_id, T)
    tok_ref[...] = jnp.where(gs_ref[...] > 0.0, tokf.astype(jnp.int32), filler)


def _ffn_body(x_ref, w1_ref, w2_ref, w3_ref, g_ref, y_ref):
    fb = pl.program_id(1)
    xb = x_ref[0]
    h1 = lax.dot_general(xb, w1_ref[0], (((1,), (1,)), ((), ())),
                         preferred_element_type=jnp.float32)
    h2 = lax.dot_general(xb, w2_ref[0], (((1,), (1,)), ((), ())),
                         preferred_element_type=jnp.float32)
    act = h1 * lax.logistic(h1) * h2
    p = lax.dot_general(act, w3_ref[0], (((1,), (1,)), ((), ())),
                        preferred_element_type=jnp.float32)
    p = p * g_ref[0]

    @pl.when(fb == 0)
    def _():
        y_ref[0] = p

    @pl.when(fb > 0)
    def _():
        y_ref[0] = y_ref[0] + p


def _dispatch_body(tok_hbm, x_hbm, xe_hbm, idx_v, rows_v, sem):
    wid = lax.axis_index("s") * NC + lax.axis_index("c")
    base = wid * ROWS_B
    pltpu.sync_copy(tok_hbm.at[pl.ds(base, ROWS_B)], idx_v)
    pltpu.async_copy(x_hbm.at[idx_v], rows_v, sem).wait()
    pltpu.sync_copy(rows_v, xe_hbm.at[pl.ds(base, ROWS_B)])


def _combine_body(y_hbm, slot_hbm, out_hbm, idx_v, rows_v, sem):
    wid = lax.axis_index("s") * NC + lax.axis_index("c")
    base = wid * ROWS_C
    pltpu.sync_copy(slot_hbm.at[pl.ds(base, ROWS_C)], idx_v)
    pltpu.async_copy(y_hbm.at[idx_v], rows_v, sem).wait()
    pltpu.sync_copy(rows_v, out_hbm.at[pl.ds(base, ROWS_C)])


@functools.cache
def _sc_calls():
    mesh = plsc.VectorSubcoreMesh(
        core_axis_name="c", subcore_axis_name="s",
        num_cores=NC, num_subcores=NS)
    dispatch = functools.partial(
        pl.kernel,
        out_type=[jax.ShapeDtypeStruct((S, H), jnp.float32)],
        mesh=mesh,
        compiler_params=pltpu.CompilerParams(needs_layout_passes=False),
        scratch_types=[
            pltpu.VMEM((ROWS_B,), jnp.int32),
            pltpu.VMEM((ROWS_B, H), jnp.float32),
            pltpu.SemaphoreType.DMA,
        ],
    )(_dispatch_body)
    combine = functools.partial(
        pl.kernel,
        out_type=[jax.ShapeDtypeStruct((T, H), jnp.float32)],
        mesh=mesh,
        compiler_params=pltpu.CompilerParams(needs_layout_passes=False),
        scratch_types=[
            pltpu.VMEM((ROWS_C,), jnp.int32),
            pltpu.VMEM((ROWS_C, H), jnp.float32),
            pltpu.SemaphoreType.DMA,
        ],
    )(_combine_body)
    return dispatch, combine


def kernel(x, W1, W2, W3, Wg, bg):
    slot2, gs, tok = pl.pallas_call(
        _router_body,
        out_shape=[
            jax.ShapeDtypeStruct((1, T), jnp.int32),
            jax.ShapeDtypeStruct((E, CAP), jnp.float32),
            jax.ShapeDtypeStruct((E, CAP), jnp.int32),
        ],
    )(Wg, x, bg.reshape(E, 1))
    slot = slot2.reshape(T)

    _dispatch_call, _combine_call = _sc_calls()
    (xe,) = _dispatch_call(tok.reshape(S), x)

    y = pl.pallas_call(
        _ffn_body,
        grid=(E, NFB),
        in_specs=[
            pl.BlockSpec((1, CAP, H), lambda e, f: (e, 0, 0)),
            pl.BlockSpec((1, FB, H), lambda e, f: (e, f, 0)),
            pl.BlockSpec((1, FB, H), lambda e, f: (e, f, 0)),
            pl.BlockSpec((1, H, FB), lambda e, f: (e, 0, f)),
            pl.BlockSpec((1, CAP, 1), lambda e, f: (e, 0, 0)),
        ],
        out_specs=pl.BlockSpec((1, CAP, H), lambda e, f: (e, 0, 0)),
        out_shape=jax.ShapeDtypeStruct((E, CAP, H), jnp.float32),
    )(
        xe.reshape(E, CAP, H), W1, W2, W3, gs.reshape(E, CAP, 1)
    )

    (out,) = _combine_call(y.reshape(S, H), slot)
    return out
